# baseline (speedup 1.0000x reference)
import functools
import math

import jax
import jax.numpy as jnp
from jax import lax
from jax.experimental import pallas as pl
from jax.experimental.pallas import tpu as pltpu
from jax.experimental.pallas import tpu_sc as plsc

_NUM_EMB = 100000
_EMB_DIM = 64
_LOG_SIGMA1 = -1.0
_LOG_SIGMA2 = -7.0
_PRIOR_PI = 0.25
_SIGMA1 = math.exp(_LOG_SIGMA1)
_SIGMA2 = math.exp(_LOG_SIGMA2)

_BLOCK_ROWS = 1000


def _dense_kl_body(mu_ref, rho_ref, eps_ref, w_ref, kl_ref):
    mu = mu_ref[...]
    rho = rho_ref[...]
    eps = eps_ref[...]
    sigma = jax.nn.softplus(rho) + 1e-5
    w = mu + eps * sigma
    w_ref[...] = jnp.concatenate([w, w], axis=1)
    log_q = -jnp.log(sigma) - 0.5 * (eps * eps)
    ww = w * w
    lp1 = (math.log(_PRIOR_PI) - _LOG_SIGMA1) - ww * (0.5 / (_SIGMA1 * _SIGMA1))
    lp2 = (math.log(1.0 - _PRIOR_PI) - _LOG_SIGMA2) - ww * (0.5 / (_SIGMA2 * _SIGMA2))
    log_p = jnp.logaddexp(lp1, lp2)
    part = jnp.sum(log_q - log_p)

    @pl.when(pl.program_id(0) == 0)
    def _():
        kl_ref[0, 0] = 0.0

    kl_ref[0, 0] += part


def _dense_kl(mu, rho, eps):
    nblk = _NUM_EMB // _BLOCK_ROWS
    return pl.pallas_call(
        _dense_kl_body,
        grid=(nblk,),
        in_specs=[
            pl.BlockSpec((_BLOCK_ROWS, _EMB_DIM), lambda i: (i, 0)),
            pl.BlockSpec((_BLOCK_ROWS, _EMB_DIM), lambda i: (i, 0)),
            pl.BlockSpec((_BLOCK_ROWS, _EMB_DIM), lambda i: (i, 0)),
        ],
        out_specs=[
            pl.BlockSpec((_BLOCK_ROWS, 2 * _EMB_DIM), lambda i: (i, 0)),
            pl.BlockSpec((1, 1), lambda i: (0, 0), memory_space=pltpu.SMEM),
        ],
        out_shape=[
            jax.ShapeDtypeStruct((_NUM_EMB, 2 * _EMB_DIM), jnp.float32),
            jax.ShapeDtypeStruct((1, 1), jnp.float32),
        ],
    )(mu, rho, eps)


_BATCH = 4096
_HIST = 50
_B = _BATCH * _HIST
_NW = 32
_PER_W = _B // _NW
_BAT_W = _BATCH // _NW
_CB = 8
_CHUNK = _CB * _HIST
_NCHUNK = _BAT_W // _CB


def _gather_body(table_hbm, idx_hbm, out_hbm, idx_v, rows_v, out_v, sem):
    wid = lax.axis_index("s") * 2 + lax.axis_index("c")
    base = wid * _PER_W
    bbase = wid * _BAT_W
    pltpu.sync_copy(idx_hbm.at[pl.ds(base, _PER_W)], idx_v)

    def chunk_body(g, c):
        off = g * _CHUNK
        pltpu.async_copy(
            table_hbm.at[idx_v.at[pl.ds(off, _CHUNK)]], rows_v, sem
        ).wait()
        for b in range(_CB):

            def row_body(r, c2, b=b):
                for k in range(_EMB_DIM // 16):
                    out_v[b, r, pl.ds(16 * k, 16)] = (
                        rows_v[b * _HIST + r, pl.ds(16 * k, 16)])
                return c2

            lax.fori_loop(0, _HIST, row_body, 0)
        for b in range(_CB):
            pltpu.sync_copy(out_v.at[b], out_hbm.at[bbase + g * _CB + b])
        return c

    lax.fori_loop(0, _NCHUNK, chunk_body, 0)


def _sc_gather(table, idx):
    mesh = plsc.VectorSubcoreMesh(core_axis_name="c", subcore_axis_name="s")
    f = pl.kernel(
        _gather_body,
        mesh=mesh,
        out_type=jax.ShapeDtypeStruct((_BATCH, _HIST, _EMB_DIM), jnp.float32),
        scratch_types=[
            pltpu.VMEM((_PER_W,), jnp.int32),
            pltpu.VMEM((_CHUNK, 2 * _EMB_DIM), jnp.float32),
            pltpu.VMEM((_CB, _HIST, _EMB_DIM), jnp.float32),
            pltpu.SemaphoreType.DMA,
        ],
    )
    return f(table, idx)


def kernel(input, weight_mu, weight_rho, eps):
    idx = input.reshape(-1).astype(jnp.int32)
    w_dup, kl_acc = _dense_kl(weight_mu, weight_rho, eps)
    out = _sc_gather(w_dup, idx)
    return out, kl_acc[0, 0]


# speedup vs baseline: 1.0985x; 1.0985x over previous
import functools
import math

import jax
import jax.numpy as jnp
from jax import lax
from jax.experimental import pallas as pl
from jax.experimental.pallas import tpu as pltpu
from jax.experimental.pallas import tpu_sc as plsc

_NUM_EMB = 100000
_EMB_DIM = 64
_LOG_SIGMA1 = -1.0
_LOG_SIGMA2 = -7.0
_PRIOR_PI = 0.25
_SIGMA1 = math.exp(_LOG_SIGMA1)
_SIGMA2 = math.exp(_LOG_SIGMA2)

_BLOCK_ROWS = 4000


def _dense_kl_body(mu_ref, rho_ref, eps_ref, w_ref, kl_ref):
    mu = mu_ref[...]
    rho = rho_ref[...]
    eps = eps_ref[...]
    sigma = jax.nn.softplus(rho) + 1e-5
    w = mu + eps * sigma
    w_ref[...] = jnp.concatenate([w, w], axis=1)
    log_q = -jnp.log(sigma) - 0.5 * (eps * eps)
    ww = w * w
    lp1 = (math.log(_PRIOR_PI) - _LOG_SIGMA1) - ww * (0.5 / (_SIGMA1 * _SIGMA1))
    lp2 = (math.log(1.0 - _PRIOR_PI) - _LOG_SIGMA2) - ww * (0.5 / (_SIGMA2 * _SIGMA2))
    log_p = jnp.logaddexp(lp1, lp2)
    part = jnp.sum(log_q - log_p)

    @pl.when(pl.program_id(0) == 0)
    def _():
        kl_ref[0, 0] = 0.0

    kl_ref[0, 0] += part


def _dense_kl(mu, rho, eps):
    nblk = _NUM_EMB // _BLOCK_ROWS
    return pl.pallas_call(
        _dense_kl_body,
        grid=(nblk,),
        in_specs=[
            pl.BlockSpec((_BLOCK_ROWS, _EMB_DIM), lambda i: (i, 0)),
            pl.BlockSpec((_BLOCK_ROWS, _EMB_DIM), lambda i: (i, 0)),
            pl.BlockSpec((_BLOCK_ROWS, _EMB_DIM), lambda i: (i, 0)),
        ],
        out_specs=[
            pl.BlockSpec((_BLOCK_ROWS, 2 * _EMB_DIM), lambda i: (i, 0)),
            pl.BlockSpec((1, 1), lambda i: (0, 0), memory_space=pltpu.SMEM),
        ],
        out_shape=[
            jax.ShapeDtypeStruct((_NUM_EMB, 2 * _EMB_DIM), jnp.float32),
            jax.ShapeDtypeStruct((1, 1), jnp.float32),
        ],
    )(mu, rho, eps)


_BATCH = 4096
_HIST = 50
_B = _BATCH * _HIST
_NW = 32
_PER_W = _B // _NW
_BAT_W = _BATCH // _NW
_CB = 8
_CHUNK = _CB * _HIST
_NCHUNK = _BAT_W // _CB


def _gather_body(table_hbm, idx_hbm, out_hbm, idx_v, rows_v, out_v, sem):
    wid = lax.axis_index("s") * 2 + lax.axis_index("c")
    base = wid * _PER_W
    bbase = wid * _BAT_W
    pltpu.sync_copy(idx_hbm.at[pl.ds(base, _PER_W)], idx_v)

    def chunk_body(g, c):
        off = g * _CHUNK
        pltpu.async_copy(
            table_hbm.at[idx_v.at[pl.ds(off, _CHUNK)]], rows_v, sem
        ).wait()
        for b in range(_CB):

            def row_body(r, c2, b=b):
                for k in range(_EMB_DIM // 16):
                    out_v[b, r, pl.ds(16 * k, 16)] = (
                        rows_v[b * _HIST + r, pl.ds(16 * k, 16)])
                return c2

            lax.fori_loop(0, _HIST, row_body, 0)
        for b in range(_CB):
            pltpu.sync_copy(out_v.at[b], out_hbm.at[bbase + g * _CB + b])
        return c

    lax.fori_loop(0, _NCHUNK, chunk_body, 0)


def _sc_gather(table, idx):
    mesh = plsc.VectorSubcoreMesh(core_axis_name="c", subcore_axis_name="s")
    f = pl.kernel(
        _gather_body,
        mesh=mesh,
        out_type=jax.ShapeDtypeStruct((_BATCH, _HIST, _EMB_DIM), jnp.float32),
        scratch_types=[
            pltpu.VMEM((_PER_W,), jnp.int32),
            pltpu.VMEM((_CHUNK, 2 * _EMB_DIM), jnp.float32),
            pltpu.VMEM((_CB, _HIST, _EMB_DIM), jnp.float32),
            pltpu.SemaphoreType.DMA,
        ],
    )
    return f(table, idx)


def kernel(input, weight_mu, weight_rho, eps):
    idx = input.reshape(-1).astype(jnp.int32)
    w_dup, kl_acc = _dense_kl(weight_mu, weight_rho, eps)
    out = _sc_gather(w_dup, idx)
    return out, kl_acc[0, 0]


# speedup vs baseline: 1.0997x; 1.0011x over previous
import functools
import math

import jax
import jax.numpy as jnp
from jax import lax
from jax.experimental import pallas as pl
from jax.experimental.pallas import tpu as pltpu
from jax.experimental.pallas import tpu_sc as plsc

_NUM_EMB = 100000
_EMB_DIM = 64
_LOG_SIGMA1 = -1.0
_LOG_SIGMA2 = -7.0
_PRIOR_PI = 0.25
_SIGMA1 = math.exp(_LOG_SIGMA1)
_SIGMA2 = math.exp(_LOG_SIGMA2)

_BLOCK_ROWS = 10000


def _dense_kl_body(mu_ref, rho_ref, eps_ref, w_ref, kl_ref):
    mu = mu_ref[...]
    rho = rho_ref[...]
    eps = eps_ref[...]
    sigma = jax.nn.softplus(rho) + 1e-5
    w = mu + eps * sigma
    w_ref[...] = jnp.concatenate([w, w], axis=1)
    log_q = -jnp.log(sigma) - 0.5 * (eps * eps)
    ww = w * w
    lp1 = (math.log(_PRIOR_PI) - _LOG_SIGMA1) - ww * (0.5 / (_SIGMA1 * _SIGMA1))
    lp2 = (math.log(1.0 - _PRIOR_PI) - _LOG_SIGMA2) - ww * (0.5 / (_SIGMA2 * _SIGMA2))
    log_p = jnp.logaddexp(lp1, lp2)
    part = jnp.sum(log_q - log_p)

    @pl.when(pl.program_id(0) == 0)
    def _():
        kl_ref[0, 0] = 0.0

    kl_ref[0, 0] += part


def _dense_kl(mu, rho, eps):
    nblk = _NUM_EMB // _BLOCK_ROWS
    return pl.pallas_call(
        _dense_kl_body,
        grid=(nblk,),
        in_specs=[
            pl.BlockSpec((_BLOCK_ROWS, _EMB_DIM), lambda i: (i, 0)),
            pl.BlockSpec((_BLOCK_ROWS, _EMB_DIM), lambda i: (i, 0)),
            pl.BlockSpec((_BLOCK_ROWS, _EMB_DIM), lambda i: (i, 0)),
        ],
        out_specs=[
            pl.BlockSpec((_BLOCK_ROWS, 2 * _EMB_DIM), lambda i: (i, 0)),
            pl.BlockSpec((1, 1), lambda i: (0, 0), memory_space=pltpu.SMEM),
        ],
        out_shape=[
            jax.ShapeDtypeStruct((_NUM_EMB, 2 * _EMB_DIM), jnp.float32),
            jax.ShapeDtypeStruct((1, 1), jnp.float32),
        ],
    )(mu, rho, eps)


_BATCH = 4096
_HIST = 50
_B = _BATCH * _HIST
_NW = 32
_PER_W = _B // _NW
_BAT_W = _BATCH // _NW
_CB = 8
_CHUNK = _CB * _HIST
_NCHUNK = _BAT_W // _CB


def _gather_body(table_hbm, idx_hbm, out_hbm, idx_v, rows_v, out_v, sem):
    wid = lax.axis_index("s") * 2 + lax.axis_index("c")
    base = wid * _PER_W
    bbase = wid * _BAT_W
    pltpu.sync_copy(idx_hbm.at[pl.ds(base, _PER_W)], idx_v)

    def chunk_body(g, c):
        off = g * _CHUNK
        pltpu.async_copy(
            table_hbm.at[idx_v.at[pl.ds(off, _CHUNK)]], rows_v, sem
        ).wait()
        for b in range(_CB):

            def row_body(r, c2, b=b):
                for k in range(_EMB_DIM // 16):
                    out_v[b, r, pl.ds(16 * k, 16)] = (
                        rows_v[b * _HIST + r, pl.ds(16 * k, 16)])
                return c2

            lax.fori_loop(0, _HIST, row_body, 0)
        for b in range(_CB):
            pltpu.sync_copy(out_v.at[b], out_hbm.at[bbase + g * _CB + b])
        return c

    lax.fori_loop(0, _NCHUNK, chunk_body, 0)


def _sc_gather(table, idx):
    mesh = plsc.VectorSubcoreMesh(core_axis_name="c", subcore_axis_name="s")
    f = pl.kernel(
        _gather_body,
        mesh=mesh,
        out_type=jax.ShapeDtypeStruct((_BATCH, _HIST, _EMB_DIM), jnp.float32),
        scratch_types=[
            pltpu.VMEM((_PER_W,), jnp.int32),
            pltpu.VMEM((_CHUNK, 2 * _EMB_DIM), jnp.float32),
            pltpu.VMEM((_CB, _HIST, _EMB_DIM), jnp.float32),
            pltpu.SemaphoreType.DMA,
        ],
    )
    return f(table, idx)


def kernel(input, weight_mu, weight_rho, eps):
    idx = input.reshape(-1).astype(jnp.int32)
    w_dup, kl_acc = _dense_kl(weight_mu, weight_rho, eps)
    out = _sc_gather(w_dup, idx)
    return out, kl_acc[0, 0]


# speedup vs baseline: 1.1893x; 1.0814x over previous
import functools
import math

import jax
import jax.numpy as jnp
from jax import lax
from jax.experimental import pallas as pl
from jax.experimental.pallas import tpu as pltpu
from jax.experimental.pallas import tpu_sc as plsc

_NUM_EMB = 100000
_EMB_DIM = 64
_LOG_SIGMA1 = -1.0
_LOG_SIGMA2 = -7.0
_PRIOR_PI = 0.25
_SIGMA1 = math.exp(_LOG_SIGMA1)
_SIGMA2 = math.exp(_LOG_SIGMA2)

_BLOCK_ROWS = 10000


def _dense_kl_body(mu_ref, rho_ref, eps_ref, w_ref, kl_ref):
    mu = mu_ref[...]
    rho = rho_ref[...]
    eps = eps_ref[...]
    sigma = jax.nn.softplus(rho) + 1e-5
    w = mu + eps * sigma
    w_ref[...] = jnp.concatenate([w, w], axis=1)
    log_q = -jnp.log(sigma) - 0.5 * (eps * eps)
    ww = w * w
    lp1 = (math.log(_PRIOR_PI) - _LOG_SIGMA1) - ww * (0.5 / (_SIGMA1 * _SIGMA1))
    lp2 = (math.log(1.0 - _PRIOR_PI) - _LOG_SIGMA2) - ww * (0.5 / (_SIGMA2 * _SIGMA2))
    log_p = jnp.logaddexp(lp1, lp2)
    part = jnp.sum(log_q - log_p)

    @pl.when(pl.program_id(0) == 0)
    def _():
        kl_ref[0, 0] = 0.0

    kl_ref[0, 0] += part


def _dense_kl(mu, rho, eps):
    nblk = _NUM_EMB // _BLOCK_ROWS
    return pl.pallas_call(
        _dense_kl_body,
        grid=(nblk,),
        in_specs=[
            pl.BlockSpec((_BLOCK_ROWS, _EMB_DIM), lambda i: (i, 0)),
            pl.BlockSpec((_BLOCK_ROWS, _EMB_DIM), lambda i: (i, 0)),
            pl.BlockSpec((_BLOCK_ROWS, _EMB_DIM), lambda i: (i, 0)),
        ],
        out_specs=[
            pl.BlockSpec((_BLOCK_ROWS, 2 * _EMB_DIM), lambda i: (i, 0)),
            pl.BlockSpec((1, 1), lambda i: (0, 0), memory_space=pltpu.SMEM),
        ],
        out_shape=[
            jax.ShapeDtypeStruct((_NUM_EMB, 2 * _EMB_DIM), jnp.float32),
            jax.ShapeDtypeStruct((1, 1), jnp.float32),
        ],
    )(mu, rho, eps)


_BATCH = 4096
_HIST = 50
_B = _BATCH * _HIST
_NW = 32
_PER_W = _B // _NW
_BAT_W = _BATCH // _NW
_CB = 4
_CHUNK = _CB * _HIST
_NCHUNK = _BAT_W // _CB


def _gather_body(table_hbm, idx_hbm, out_hbm, idx_v,
                 rows0, rows1, out0, out1, gsem0, gsem1, wsem0, wsem1):
    wid = lax.axis_index("s") * 2 + lax.axis_index("c")
    base = wid * _PER_W
    bbase = wid * _BAT_W
    pltpu.sync_copy(idx_hbm.at[pl.ds(base, _PER_W)], idx_v)
    rows = (rows0, rows1)
    outs = (out0, out1)
    gsems = (gsem0, gsem1)
    wsems = (wsem0, wsem1)

    def start_gather(g, buf):
        return pltpu.async_copy(
            table_hbm.at[idx_v.at[pl.ds(g * _CHUNK, _CHUNK)]],
            rows[buf], gsems[buf])

    gcopy = [start_gather(0, 0), start_gather(1, 1)]
    wcopy = [None, None]
    for g in range(_NCHUNK):
        p = g & 1
        gcopy[p].wait()
        if wcopy[p] is not None:
            for c in wcopy[p]:
                c.wait()
        for b in range(_CB):

            def row_body(r, c2, b=b, p=p):
                for k in range(_EMB_DIM // 16):
                    outs[p][b, r, pl.ds(16 * k, 16)] = (
                        rows[p][b * _HIST + r, pl.ds(16 * k, 16)])
                return c2

            lax.fori_loop(0, _HIST, row_body, 0)
        if g + 2 < _NCHUNK:
            gcopy[p] = start_gather(g + 2, p)
        wcopy[p] = [
            pltpu.async_copy(outs[p].at[b], out_hbm.at[bbase + g * _CB + b],
                             wsems[p])
            for b in range(_CB)
        ]
    for p in range(2):
        for c in wcopy[p]:
            c.wait()


def _sc_gather(table, idx):
    mesh = plsc.VectorSubcoreMesh(core_axis_name="c", subcore_axis_name="s")
    f = pl.kernel(
        _gather_body,
        mesh=mesh,
        out_type=jax.ShapeDtypeStruct((_BATCH, _HIST, _EMB_DIM), jnp.float32),
        scratch_types=[
            pltpu.VMEM((_PER_W,), jnp.int32),
            pltpu.VMEM((_CHUNK, 2 * _EMB_DIM), jnp.float32),
            pltpu.VMEM((_CHUNK, 2 * _EMB_DIM), jnp.float32),
            pltpu.VMEM((_CB, _HIST, _EMB_DIM), jnp.float32),
            pltpu.VMEM((_CB, _HIST, _EMB_DIM), jnp.float32),
            pltpu.SemaphoreType.DMA,
            pltpu.SemaphoreType.DMA,
            pltpu.SemaphoreType.DMA,
            pltpu.SemaphoreType.DMA,
        ],
    )
    return f(table, idx)


def kernel(input, weight_mu, weight_rho, eps):
    idx = input.reshape(-1).astype(jnp.int32)
    w_dup, kl_acc = _dense_kl(weight_mu, weight_rho, eps)
    out = _sc_gather(w_dup, idx)
    return out, kl_acc[0, 0]
